# baseline (speedup 1.0000x reference)
import functools

import jax
import jax.numpy as jnp
from jax import lax
from jax.experimental import pallas as pl
from jax.experimental.pallas import tpu as pltpu
from jax.experimental.pallas import tpu_sc as plsc

_NC = 2
_NS = 16
_RB = 384
_ZB = 64

_pallas_call = pl.pallas_call
_BT = 2000
_SC_PARAMS = pltpu.CompilerParams(use_tc_tiling_on_sc=False)


def _round_up(v, m):
    return (v + m - 1) // m * m


def _sc_degree(nr, epr, src_rows):
    rt = epr // (_NS * _NC)
    nrt = nr // _NS
    nzc = nrt // _ZB
    mesh = plsc.VectorSubcoreMesh(core_axis_name="c", subcore_axis_name="s",
                                  num_cores=_NC)

    def body(zeros_h, ones_h, src_h, out0, out1, accum, sidx, obuf, zbuf,
             ssem0, ssem1):
        c = lax.axis_index("c")
        s = lax.axis_index("s")
        pltpu.sync_copy(zeros_h, zbuf)

        def zstep(j, carry):
            pltpu.sync_copy(zbuf, accum.at[pl.ds(s * nrt + j * _ZB, _ZB)])
            return carry

        lax.fori_loop(0, nzc, zstep, 0)
        plsc.subcore_barrier()

        pltpu.sync_copy(ones_h, obuf)
        ebase = (c * _NS + s) * rt
        ssems = (ssem0, ssem1)

        def gstart(j, b):
            pltpu.sync_copy(src_h.at[j], sidx.at[b])
            pltpu.async_copy(obuf, accum.at[sidx.at[b]], ssems[b], add=True)

        def swait(b):
            pltpu.make_async_copy(obuf, accum.at[sidx.at[b]],
                                  ssems[b]).wait()

        gstart(ebase, 0)
        gstart(ebase + 1, 1)

        def estep(u, carry):
            i0 = ebase + 2 * u + 2
            swait(0)
            gstart(i0, 0)
            swait(1)
            gstart(i0 + 1, 1)
            return carry

        lax.fori_loop(0, (rt - 2) // 2, estep, 0)
        swait(0)
        swait(1)
        plsc.subcore_barrier()

        @pl.when(c == 0)
        def _():
            pltpu.sync_copy(accum.at[pl.ds(s * nrt, nrt)],
                            out0.at[pl.ds(s * nrt, nrt)])

        @pl.when(c == 1)
        def _():
            pltpu.sync_copy(accum.at[pl.ds(s * nrt, nrt)],
                            out1.at[pl.ds(s * nrt, nrt)])

    fn = pl.kernel(
        body,
        out_type=[jax.ShapeDtypeStruct((nr, 8), jnp.float32),
                  jax.ShapeDtypeStruct((nr, 8), jnp.float32)],
        mesh=mesh,
        scratch_types=[
            pltpu.VMEM_SHARED((nr, 8), jnp.float32),
            pltpu.VMEM((2, _RB), jnp.int32),
            pltpu.VMEM((_RB, 8), jnp.float32),
            pltpu.VMEM((_ZB, 8), jnp.float32),
            pltpu.SemaphoreType.DMA,
            pltpu.SemaphoreType.DMA,
        ],
        compiler_params=_SC_PARAMS,
    )
    zeros_h = jnp.zeros((_ZB, 8), jnp.float32)
    ones_h = jnp.ones((_RB, 8), jnp.float32)
    return fn(zeros_h, ones_h, src_rows)


def _sc_segsum(tab0, tab1, sd_rows, nr, epr, width, feature_split):
    rt = epr // _NS if feature_split else epr // (_NS * _NC)
    nrt = nr // _NS
    nzc = nrt // _ZB
    mesh = plsc.VectorSubcoreMesh(core_axis_name="c", subcore_axis_name="s",
                                  num_cores=_NC)

    def body(zeros_h, tab0_h, tab1_h, sd_h, out0, out1,
             accum, sd, rows, zbuf, gsem0, gsem1, ssem0, ssem1):
        c = lax.axis_index("c")
        s = lax.axis_index("s")
        pltpu.sync_copy(zeros_h, zbuf)

        def zstep(j, carry):
            pltpu.sync_copy(zbuf, accum.at[pl.ds(s * nrt + j * _ZB, _ZB)])
            return carry

        lax.fori_loop(0, nzc, zstep, 0)
        plsc.subcore_barrier()

        ebase = (s * rt) if feature_split else ((c * _NS + s) * rt)
        gsems = (gsem0, gsem1)
        ssems = (ssem0, ssem1)

        def gstart(j, b):
            pltpu.sync_copy(sd_h.at[j], sd.at[b])
            if feature_split:
                @pl.when(c == 0)
                def _():
                    pltpu.async_copy(tab0_h.at[sd.at[b, 0]], rows.at[b],
                                     gsems[b])

                @pl.when(c == 1)
                def _():
                    pltpu.async_copy(tab1_h.at[sd.at[b, 0]], rows.at[b],
                                     gsems[b])
            else:
                pltpu.async_copy(tab0_h.at[sd.at[b, 0]], rows.at[b],
                                 gsems[b])

        def gwait(b):
            pltpu.make_async_copy(tab0_h.at[sd.at[b, 0]], rows.at[b],
                                  gsems[b]).wait()

        def sstart(b):
            pltpu.async_copy(rows.at[b], accum.at[sd.at[b, 1]], ssems[b],
                             add=True)

        def swait(b):
            pltpu.make_async_copy(rows.at[b], accum.at[sd.at[b, 1]],
                                  ssems[b]).wait()

        gstart(ebase, 0)
        gstart(ebase + 1, 1)
        gwait(0)
        sstart(0)

        def mstep(u, carry):
            i0 = ebase + 2 * u + 2
            swait(0)
            gstart(i0, 0)
            gwait(1)
            sstart(1)
            swait(1)
            gstart(i0 + 1, 1)
            gwait(0)
            sstart(0)
            return carry

        lax.fori_loop(0, (rt - 2) // 2, mstep, 0)
        gwait(1)
        sstart(1)
        swait(0)
        swait(1)

        plsc.subcore_barrier()

        @pl.when(c == 0)
        def _():
            pltpu.sync_copy(accum.at[pl.ds(s * nrt, nrt)],
                            out0.at[pl.ds(s * nrt, nrt)])

        @pl.when(c == 1)
        def _():
            pltpu.sync_copy(accum.at[pl.ds(s * nrt, nrt)],
                            out1.at[pl.ds(s * nrt, nrt)])

    fn = pl.kernel(
        body,
        out_type=[jax.ShapeDtypeStruct((nr, width), jnp.float32),
                  jax.ShapeDtypeStruct((nr, width), jnp.float32)],
        mesh=mesh,
        scratch_types=[
            pltpu.VMEM_SHARED((nr, width), jnp.float32),
            pltpu.VMEM((2, 2, _RB), jnp.int32),
            pltpu.VMEM((2, _RB, width), jnp.float32),
            pltpu.VMEM((_ZB, width), jnp.float32),
            pltpu.SemaphoreType.DMA,
            pltpu.SemaphoreType.DMA,
            pltpu.SemaphoreType.DMA,
            pltpu.SemaphoreType.DMA,
        ],
        compiler_params=_SC_PARAMS,
    )
    zeros_h = jnp.zeros((_ZB, width), jnp.float32)
    return fn(zeros_h, tab0, tab1, sd_rows)


def _tc_prep(h16, d0, d1, n, nr):
    def body(h_ref, d0_ref, d1_ref, dis_ref, th_ref):
        deg = d0_ref[:, 0:1] + d1_ref[:, 0:1]
        dis = jnp.where(deg > 0, lax.rsqrt(jnp.maximum(deg, 1e-12)), 0.0)
        dis_ref[...] = jnp.broadcast_to(dis, (_BT, 8))
        th_ref[...] = h_ref[...] * dis

    return _pallas_call(
        body,
        grid=(n // _BT,),
        in_specs=[
            pl.BlockSpec((_BT, 16), lambda i: (i, 0)),
            pl.BlockSpec((_BT, 8), lambda i: (i, 0)),
            pl.BlockSpec((_BT, 8), lambda i: (i, 0)),
        ],
        out_specs=[
            pl.BlockSpec((_BT, 8), lambda i: (i, 0)),
            pl.BlockSpec((_BT, 16), lambda i: (i, 0)),
        ],
        out_shape=[
            jax.ShapeDtypeStruct((n, 8), jnp.float32),
            jax.ShapeDtypeStruct((nr, 16), jnp.float32),
        ],
    )(h16, d0, d1)


def _tc_th1_16(s1a, s1b, dis8, n, nr):
    def body(a_ref, b_ref, dis_ref, th_ref):
        dis = dis_ref[:, 0:1]
        th_ref[...] = (-dis * dis) * (a_ref[...] + b_ref[...])

    return _pallas_call(
        body,
        grid=(n // _BT,),
        in_specs=[
            pl.BlockSpec((_BT, 16), lambda i: (i, 0)),
            pl.BlockSpec((_BT, 16), lambda i: (i, 0)),
            pl.BlockSpec((_BT, 8), lambda i: (i, 0)),
        ],
        out_specs=pl.BlockSpec((_BT, 16), lambda i: (i, 0)),
        out_shape=jax.ShapeDtypeStruct((nr, 16), jnp.float32),
    )(s1a, s1b, dis8)


def _tc_th1_64(s1a, s1b, dis8, n, nr):
    def body(a_ref, b_ref, dis_ref, ta_ref, tb_ref):
        dis = dis_ref[:, 0:1]
        nd2 = -dis * dis
        ta_ref[...] = nd2 * a_ref[...]
        tb_ref[...] = nd2 * b_ref[...]

    return _pallas_call(
        body,
        grid=(n // _BT,),
        in_specs=[
            pl.BlockSpec((_BT, 32), lambda i: (i, 0)),
            pl.BlockSpec((_BT, 32), lambda i: (i, 0)),
            pl.BlockSpec((_BT, 8), lambda i: (i, 0)),
        ],
        out_specs=[
            pl.BlockSpec((_BT, 32), lambda i: (i, 0)),
            pl.BlockSpec((_BT, 32), lambda i: (i, 0)),
        ],
        out_shape=[
            jax.ShapeDtypeStruct((nr, 32), jnp.float32),
            jax.ShapeDtypeStruct((nr, 32), jnp.float32),
        ],
    )(s1a, s1b, dis8)


def _tc_cheb_a(tx0, s1a, s1b, dis8, w0m2, w1, b, n, partial_mode):
    din = w0m2.shape[0]
    dout = w0m2.shape[1]
    hw = s1a.shape[1]

    def body(tx0_ref, s1a_ref, s1b_ref, dis_ref, w0_ref, w1_ref, b_ref,
             yp_ref):
        dis = dis_ref[:, 0:1]
        if partial_mode:
            s1 = s1a_ref[...] + s1b_ref[...]
        else:
            s1 = jnp.concatenate([s1a_ref[...], s1b_ref[...]], axis=1)
        tx1 = (-dis) * s1
        yp_ref[...] = (
            jnp.dot(tx0_ref[...], w0_ref[...],
                    preferred_element_type=jnp.float32)
            + jnp.dot(tx1, w1_ref[...], preferred_element_type=jnp.float32)
            + b_ref[...])

    return _pallas_call(
        body,
        grid=(n // _BT,),
        in_specs=[
            pl.BlockSpec((_BT, din), lambda i: (i, 0)),
            pl.BlockSpec((_BT, hw), lambda i: (i, 0)),
            pl.BlockSpec((_BT, hw), lambda i: (i, 0)),
            pl.BlockSpec((_BT, 8), lambda i: (i, 0)),
            pl.BlockSpec((din, dout), lambda i: (0, 0)),
            pl.BlockSpec((din, dout), lambda i: (0, 0)),
            pl.BlockSpec((1, dout), lambda i: (0, 0)),
        ],
        out_specs=pl.BlockSpec((_BT, dout), lambda i: (i, 0)),
        out_shape=jax.ShapeDtypeStruct((n, dout), jnp.float32),
    )(tx0, s1a, s1b, dis8, w0m2, w1, b)


def _tc_cheb_b(ypart, s2a, s2b, dis8, w2, a, n, partial_mode):
    dout = w2.shape[1]
    hw = s2a.shape[1]

    def body(yp_ref, s2a_ref, s2b_ref, dis_ref, w2_ref, a_ref,
             p_ref, st_ref):
        i = pl.program_id(0)
        dis = dis_ref[:, 0:1]
        if partial_mode:
            s2 = s2a_ref[...] + s2b_ref[...]
        else:
            s2 = jnp.concatenate([s2a_ref[...], s2b_ref[...]], axis=1)
        t2 = (-2.0 * dis) * s2
        y = yp_ref[...] + jnp.dot(t2, w2_ref[...],
                                  preferred_element_type=jnp.float32)
        aa = a_ref[0, 0]
        p = jnp.where(y > 0, y, aa * y)
        p_ref[...] = p

        @pl.when(i == 0)
        def _():
            st_ref[...] = jnp.zeros_like(st_ref)

        st_ref[0:1, :] += jnp.sum(p, axis=0, keepdims=True)
        st_ref[1:2, :] += jnp.sum(p * p, axis=0, keepdims=True)

    return _pallas_call(
        body,
        grid=(n // _BT,),
        in_specs=[
            pl.BlockSpec((_BT, dout), lambda i: (i, 0)),
            pl.BlockSpec((_BT, hw), lambda i: (i, 0)),
            pl.BlockSpec((_BT, hw), lambda i: (i, 0)),
            pl.BlockSpec((_BT, 8), lambda i: (i, 0)),
            pl.BlockSpec((hw * 2 if not partial_mode else hw, dout),
                         lambda i: (0, 0)),
            pl.BlockSpec((1, 1), lambda i: (0, 0)),
        ],
        out_specs=[
            pl.BlockSpec((_BT, dout), lambda i: (i, 0)),
            pl.BlockSpec((2, dout), lambda i: (0, 0)),
        ],
        out_shape=[
            jax.ShapeDtypeStruct((n, dout), jnp.float32),
            jax.ShapeDtypeStruct((2, dout), jnp.float32),
        ],
    )(ypart, s2a, s2b, dis8, w2, a)


def _tc_bn(p, st, g, bt, dis8, n, nr, last):
    dout = p.shape[1]
    inv_n = 1.0 / n

    def body(p_ref, st_ref, g_ref, bt_ref, dis_ref, *outs):
        mean = st_ref[0:1, :] * inv_n
        var = st_ref[1:2, :] * inv_n - mean * mean
        scale = lax.rsqrt(var + 1e-5) * g_ref[...]
        y = (p_ref[...] - mean) * scale + bt_ref[...]
        outs[0][...] = y
        if not last:
            dis = dis_ref[:, 0:1]
            th = dis * y
            outs[1][...] = th[:, :32]
            outs[2][...] = th[:, 32:]

    out_specs = [pl.BlockSpec((_BT, dout), lambda i: (i, 0))]
    out_shape = [jax.ShapeDtypeStruct((n, dout), jnp.float32)]
    if not last:
        out_specs += [pl.BlockSpec((_BT, 32), lambda i: (i, 0)),
                      pl.BlockSpec((_BT, 32), lambda i: (i, 0))]
        out_shape += [jax.ShapeDtypeStruct((nr, 32), jnp.float32),
                      jax.ShapeDtypeStruct((nr, 32), jnp.float32)]

    return _pallas_call(
        body,
        grid=(n // _BT,),
        in_specs=[
            pl.BlockSpec((_BT, dout), lambda i: (i, 0)),
            pl.BlockSpec((2, dout), lambda i: (0, 0)),
            pl.BlockSpec((1, dout), lambda i: (0, 0)),
            pl.BlockSpec((1, dout), lambda i: (0, 0)),
            pl.BlockSpec((_BT, 8), lambda i: (i, 0)),
        ],
        out_specs=out_specs,
        out_shape=out_shape,
    )(p, st, g.reshape(1, -1), bt.reshape(1, -1), dis8)


def kernel(x, pos, normals, edge_index,
           W1, b1, a1, g1, bt1,
           W2, b2, a2, g2, bt2,
           W3, b3, a3, g3, bt3,
           W4, b4, a4, g4, bt4):
    n = x.shape[0]
    e = edge_index.shape[1]
    nr = _round_up(n + 1, _NS * _ZB)
    ep = _round_up(e, _NC * _NS * _RB)
    epr = ep // _RB

    src = edge_index[0].astype(jnp.int32)
    dst = edge_index[1].astype(jnp.int32)
    padk = ep - e
    phantom = jnp.full((padk,), n, jnp.int32)
    srcp = jnp.concatenate([src, phantom]).reshape(epr, _RB)
    dstp = jnp.concatenate([dst, phantom]).reshape(epr, _RB)
    sd = jnp.stack([srcp, dstp], axis=1)

    deg0, deg1 = _sc_degree(nr, epr, srcp)

    h16 = jnp.pad(jnp.concatenate([x, pos, normals], axis=1),
                  ((0, 0), (0, 7)))
    dis8, th0 = _tc_prep(h16, deg0, deg1, n, nr)

    s1a, s1b = _sc_segsum(th0, th0, sd, nr, epr, 16, False)
    th1 = _tc_th1_16(s1a, s1b, dis8, n, nr)
    s2a, s2b = _sc_segsum(th1, th1, sd, nr, epr, 16, False)
    w1p = jnp.pad(W1, ((0, 0), (0, 7), (0, 0)))
    yp = _tc_cheb_a(h16, s1a, s1b, dis8, w1p[0] - w1p[2], w1p[1],
                    b1.reshape(1, -1), n, True)
    p, st = _tc_cheb_b(yp, s2a, s2b, dis8, w1p[2],
                       a1.reshape(1, 1), n, True)
    tx0, tha, thb = _tc_bn(p, st, g1, bt1, dis8, n, nr, False)

    for (W, b, a, g, bt, last) in (
            (W2, b2, a2, g2, bt2, False),
            (W3, b3, a3, g3, bt3, False),
            (W4, b4, a4, g4, bt4, True)):
        s1a, s1b = _sc_segsum(tha, thb, sd, nr, epr, 32, True)
        th1a, th1b = _tc_th1_64(s1a, s1b, dis8, n, nr)
        s2a, s2b = _sc_segsum(th1a, th1b, sd, nr, epr, 32, True)
        yp = _tc_cheb_a(tx0, s1a, s1b, dis8, W[0] - W[2], W[1],
                        b.reshape(1, -1), n, False)
        p, st = _tc_cheb_b(yp, s2a, s2b, dis8, W[2],
                           a.reshape(1, 1), n, False)
        bn = _tc_bn(p, st, g, bt, dis8, n, nr, last)
        if last:
            return bn[0]
        tx0, tha, thb = bn


# speedup vs baseline: 1.1892x; 1.1892x over previous
import functools

import jax
import jax.numpy as jnp
from jax import lax
from jax.experimental import pallas as pl
from jax.experimental.pallas import tpu as pltpu
from jax.experimental.pallas import tpu_sc as plsc

_NC = 2
_NS = 16
_RB = 256
_ZB = 128

_pallas_call = pl.pallas_call
_BT = 2000
_SC_PARAMS = pltpu.CompilerParams(use_tc_tiling_on_sc=False)


def _round_up(v, m):
    return (v + m - 1) // m * m


def _sc_degree(nr, epr, src_rows):
    rt = epr // (_NS * _NC)
    nrt = nr // _NS
    nzc = nrt // _ZB
    mesh = plsc.VectorSubcoreMesh(core_axis_name="c", subcore_axis_name="s",
                                  num_cores=_NC)

    def body(zeros_h, ones_h, src_h, out0, out1, accum, sidx, obuf, zbuf,
             ssem0, ssem1):
        c = lax.axis_index("c")
        s = lax.axis_index("s")
        pltpu.sync_copy(zeros_h, zbuf)

        def zstep(j, carry):
            pltpu.sync_copy(zbuf, accum.at[pl.ds(s * nrt + j * _ZB, _ZB)])
            return carry

        lax.fori_loop(0, nzc, zstep, 0)
        plsc.subcore_barrier()

        pltpu.sync_copy(ones_h, obuf)
        ebase = (c * _NS + s) * rt
        ssems = (ssem0, ssem1)

        def gstart(j, b):
            pltpu.sync_copy(src_h.at[j], sidx.at[b])
            pltpu.async_copy(obuf, accum.at[sidx.at[b]], ssems[b], add=True)

        def swait(b):
            pltpu.make_async_copy(obuf, accum.at[sidx.at[b]],
                                  ssems[b]).wait()

        gstart(ebase, 0)
        gstart(ebase + 1, 1)

        def estep(u, carry):
            i0 = ebase + 2 * u + 2
            swait(0)
            gstart(i0, 0)
            swait(1)
            gstart(i0 + 1, 1)
            return carry

        lax.fori_loop(0, (rt - 2) // 2, estep, 0)
        swait(0)
        swait(1)
        plsc.subcore_barrier()

        @pl.when(c == 0)
        def _():
            pltpu.sync_copy(accum.at[pl.ds(s * nrt, nrt)],
                            out0.at[pl.ds(s * nrt, nrt)])

        @pl.when(c == 1)
        def _():
            pltpu.sync_copy(accum.at[pl.ds(s * nrt, nrt)],
                            out1.at[pl.ds(s * nrt, nrt)])

    fn = pl.kernel(
        body,
        out_type=[jax.ShapeDtypeStruct((nr, 8), jnp.float32),
                  jax.ShapeDtypeStruct((nr, 8), jnp.float32)],
        mesh=mesh,
        scratch_types=[
            pltpu.VMEM_SHARED((nr, 8), jnp.float32),
            pltpu.VMEM((2, _RB), jnp.int32),
            pltpu.VMEM((_RB, 8), jnp.float32),
            pltpu.VMEM((_ZB, 8), jnp.float32),
            pltpu.SemaphoreType.DMA,
            pltpu.SemaphoreType.DMA,
        ],
        compiler_params=_SC_PARAMS,
    )
    zeros_h = jnp.zeros((_ZB, 8), jnp.float32)
    ones_h = jnp.ones((_RB, 8), jnp.float32)
    return fn(zeros_h, ones_h, src_rows)


def _sc_segsum(tab0, tab1, sd_rows, nr, epr, width, feature_split):
    rt = epr // _NS if feature_split else epr // (_NS * _NC)
    nrt = nr // _NS
    nzc = nrt // _ZB
    mesh = plsc.VectorSubcoreMesh(core_axis_name="c", subcore_axis_name="s",
                                  num_cores=_NC)

    def body(zeros_h, tab0_h, tab1_h, sd_h, out0, out1,
             accum, sd, rows, zbuf, gsem0, gsem1, ssem0, ssem1):
        c = lax.axis_index("c")
        s = lax.axis_index("s")
        pltpu.sync_copy(zeros_h, zbuf)

        def zstep(j, carry):
            pltpu.sync_copy(zbuf, accum.at[pl.ds(s * nrt + j * _ZB, _ZB)])
            return carry

        lax.fori_loop(0, nzc, zstep, 0)
        plsc.subcore_barrier()

        ebase = (s * rt) if feature_split else ((c * _NS + s) * rt)
        gsems = (gsem0, gsem1)
        ssems = (ssem0, ssem1)

        def gstart(j, b):
            pltpu.sync_copy(sd_h.at[j], sd.at[b])
            if feature_split:
                @pl.when(c == 0)
                def _():
                    pltpu.async_copy(tab0_h.at[sd.at[b, 0]], rows.at[b],
                                     gsems[b])

                @pl.when(c == 1)
                def _():
                    pltpu.async_copy(tab1_h.at[sd.at[b, 0]], rows.at[b],
                                     gsems[b])
            else:
                pltpu.async_copy(tab0_h.at[sd.at[b, 0]], rows.at[b],
                                 gsems[b])

        def gwait(b):
            pltpu.make_async_copy(tab0_h.at[sd.at[b, 0]], rows.at[b],
                                  gsems[b]).wait()

        def sstart(b):
            pltpu.async_copy(rows.at[b], accum.at[sd.at[b, 1]], ssems[b],
                             add=True)

        def swait(b):
            pltpu.make_async_copy(rows.at[b], accum.at[sd.at[b, 1]],
                                  ssems[b]).wait()

        gstart(ebase, 0)
        gstart(ebase + 1, 1)
        gwait(0)
        sstart(0)

        def mstep(u, carry):
            i0 = ebase + 2 * u + 2
            swait(0)
            gstart(i0, 0)
            gwait(1)
            sstart(1)
            swait(1)
            gstart(i0 + 1, 1)
            gwait(0)
            sstart(0)
            return carry

        lax.fori_loop(0, (rt - 2) // 2, mstep, 0)
        gwait(1)
        sstart(1)
        swait(0)
        swait(1)

        plsc.subcore_barrier()

        @pl.when(c == 0)
        def _():
            pltpu.sync_copy(accum.at[pl.ds(s * nrt, nrt)],
                            out0.at[pl.ds(s * nrt, nrt)])

        @pl.when(c == 1)
        def _():
            pltpu.sync_copy(accum.at[pl.ds(s * nrt, nrt)],
                            out1.at[pl.ds(s * nrt, nrt)])

    fn = pl.kernel(
        body,
        out_type=[jax.ShapeDtypeStruct((nr, width), jnp.float32),
                  jax.ShapeDtypeStruct((nr, width), jnp.float32)],
        mesh=mesh,
        scratch_types=[
            pltpu.VMEM_SHARED((nr, width), jnp.float32),
            pltpu.VMEM((2, 2, _RB), jnp.int32),
            pltpu.VMEM((2, _RB, width), jnp.float32),
            pltpu.VMEM((_ZB, width), jnp.float32),
            pltpu.SemaphoreType.DMA,
            pltpu.SemaphoreType.DMA,
            pltpu.SemaphoreType.DMA,
            pltpu.SemaphoreType.DMA,
        ],
        compiler_params=_SC_PARAMS,
    )
    zeros_h = jnp.zeros((_ZB, width), jnp.float32)
    return fn(zeros_h, tab0, tab1, sd_rows)


def _tc_prep(h16, d0, d1, n, nr):
    def body(h_ref, d0_ref, d1_ref, dis_ref, th_ref):
        deg = d0_ref[:, 0:1] + d1_ref[:, 0:1]
        dis = jnp.where(deg > 0, lax.rsqrt(jnp.maximum(deg, 1e-12)), 0.0)
        dis_ref[...] = jnp.broadcast_to(dis, (_BT, 8))
        th_ref[...] = h_ref[...] * dis

    return _pallas_call(
        body,
        grid=(n // _BT,),
        in_specs=[
            pl.BlockSpec((_BT, 16), lambda i: (i, 0)),
            pl.BlockSpec((_BT, 8), lambda i: (i, 0)),
            pl.BlockSpec((_BT, 8), lambda i: (i, 0)),
        ],
        out_specs=[
            pl.BlockSpec((_BT, 8), lambda i: (i, 0)),
            pl.BlockSpec((_BT, 16), lambda i: (i, 0)),
        ],
        out_shape=[
            jax.ShapeDtypeStruct((n, 8), jnp.float32),
            jax.ShapeDtypeStruct((nr, 16), jnp.float32),
        ],
    )(h16, d0, d1)


def _tc_th1_16(s1a, s1b, dis8, n, nr):
    def body(a_ref, b_ref, dis_ref, th_ref):
        dis = dis_ref[:, 0:1]
        th_ref[...] = (-dis * dis) * (a_ref[...] + b_ref[...])

    return _pallas_call(
        body,
        grid=(n // _BT,),
        in_specs=[
            pl.BlockSpec((_BT, 16), lambda i: (i, 0)),
            pl.BlockSpec((_BT, 16), lambda i: (i, 0)),
            pl.BlockSpec((_BT, 8), lambda i: (i, 0)),
        ],
        out_specs=pl.BlockSpec((_BT, 16), lambda i: (i, 0)),
        out_shape=jax.ShapeDtypeStruct((nr, 16), jnp.float32),
    )(s1a, s1b, dis8)


def _tc_th1_64(s1a, s1b, dis8, n, nr):
    def body(a_ref, b_ref, dis_ref, ta_ref, tb_ref):
        dis = dis_ref[:, 0:1]
        nd2 = -dis * dis
        ta_ref[...] = nd2 * a_ref[...]
        tb_ref[...] = nd2 * b_ref[...]

    return _pallas_call(
        body,
        grid=(n // _BT,),
        in_specs=[
            pl.BlockSpec((_BT, 32), lambda i: (i, 0)),
            pl.BlockSpec((_BT, 32), lambda i: (i, 0)),
            pl.BlockSpec((_BT, 8), lambda i: (i, 0)),
        ],
        out_specs=[
            pl.BlockSpec((_BT, 32), lambda i: (i, 0)),
            pl.BlockSpec((_BT, 32), lambda i: (i, 0)),
        ],
        out_shape=[
            jax.ShapeDtypeStruct((nr, 32), jnp.float32),
            jax.ShapeDtypeStruct((nr, 32), jnp.float32),
        ],
    )(s1a, s1b, dis8)


def _tc_cheb_a(tx0, s1a, s1b, dis8, w0m2, w1, b, n, partial_mode):
    din = w0m2.shape[0]
    dout = w0m2.shape[1]
    hw = s1a.shape[1]

    def body(tx0_ref, s1a_ref, s1b_ref, dis_ref, w0_ref, w1_ref, b_ref,
             yp_ref):
        dis = dis_ref[:, 0:1]
        if partial_mode:
            s1 = s1a_ref[...] + s1b_ref[...]
        else:
            s1 = jnp.concatenate([s1a_ref[...], s1b_ref[...]], axis=1)
        tx1 = (-dis) * s1
        yp_ref[...] = (
            jnp.dot(tx0_ref[...], w0_ref[...],
                    preferred_element_type=jnp.float32)
            + jnp.dot(tx1, w1_ref[...], preferred_element_type=jnp.float32)
            + b_ref[...])

    return _pallas_call(
        body,
        grid=(n // _BT,),
        in_specs=[
            pl.BlockSpec((_BT, din), lambda i: (i, 0)),
            pl.BlockSpec((_BT, hw), lambda i: (i, 0)),
            pl.BlockSpec((_BT, hw), lambda i: (i, 0)),
            pl.BlockSpec((_BT, 8), lambda i: (i, 0)),
            pl.BlockSpec((din, dout), lambda i: (0, 0)),
            pl.BlockSpec((din, dout), lambda i: (0, 0)),
            pl.BlockSpec((1, dout), lambda i: (0, 0)),
        ],
        out_specs=pl.BlockSpec((_BT, dout), lambda i: (i, 0)),
        out_shape=jax.ShapeDtypeStruct((n, dout), jnp.float32),
    )(tx0, s1a, s1b, dis8, w0m2, w1, b)


def _tc_cheb_b(ypart, s2a, s2b, dis8, w2, a, n, partial_mode):
    dout = w2.shape[1]
    hw = s2a.shape[1]

    def body(yp_ref, s2a_ref, s2b_ref, dis_ref, w2_ref, a_ref,
             p_ref, st_ref):
        i = pl.program_id(0)
        dis = dis_ref[:, 0:1]
        if partial_mode:
            s2 = s2a_ref[...] + s2b_ref[...]
        else:
            s2 = jnp.concatenate([s2a_ref[...], s2b_ref[...]], axis=1)
        t2 = (-2.0 * dis) * s2
        y = yp_ref[...] + jnp.dot(t2, w2_ref[...],
                                  preferred_element_type=jnp.float32)
        aa = a_ref[0, 0]
        p = jnp.where(y > 0, y, aa * y)
        p_ref[...] = p

        @pl.when(i == 0)
        def _():
            st_ref[...] = jnp.zeros_like(st_ref)

        st_ref[0:1, :] += jnp.sum(p, axis=0, keepdims=True)
        st_ref[1:2, :] += jnp.sum(p * p, axis=0, keepdims=True)

    return _pallas_call(
        body,
        grid=(n // _BT,),
        in_specs=[
            pl.BlockSpec((_BT, dout), lambda i: (i, 0)),
            pl.BlockSpec((_BT, hw), lambda i: (i, 0)),
            pl.BlockSpec((_BT, hw), lambda i: (i, 0)),
            pl.BlockSpec((_BT, 8), lambda i: (i, 0)),
            pl.BlockSpec((hw * 2 if not partial_mode else hw, dout),
                         lambda i: (0, 0)),
            pl.BlockSpec((1, 1), lambda i: (0, 0)),
        ],
        out_specs=[
            pl.BlockSpec((_BT, dout), lambda i: (i, 0)),
            pl.BlockSpec((2, dout), lambda i: (0, 0)),
        ],
        out_shape=[
            jax.ShapeDtypeStruct((n, dout), jnp.float32),
            jax.ShapeDtypeStruct((2, dout), jnp.float32),
        ],
    )(ypart, s2a, s2b, dis8, w2, a)


def _tc_bn(p, st, g, bt, dis8, n, nr, last):
    dout = p.shape[1]
    inv_n = 1.0 / n

    def body(p_ref, st_ref, g_ref, bt_ref, dis_ref, *outs):
        mean = st_ref[0:1, :] * inv_n
        var = st_ref[1:2, :] * inv_n - mean * mean
        scale = lax.rsqrt(var + 1e-5) * g_ref[...]
        y = (p_ref[...] - mean) * scale + bt_ref[...]
        outs[0][...] = y
        if not last:
            dis = dis_ref[:, 0:1]
            th = dis * y
            outs[1][...] = th[:, :32]
            outs[2][...] = th[:, 32:]

    out_specs = [pl.BlockSpec((_BT, dout), lambda i: (i, 0))]
    out_shape = [jax.ShapeDtypeStruct((n, dout), jnp.float32)]
    if not last:
        out_specs += [pl.BlockSpec((_BT, 32), lambda i: (i, 0)),
                      pl.BlockSpec((_BT, 32), lambda i: (i, 0))]
        out_shape += [jax.ShapeDtypeStruct((nr, 32), jnp.float32),
                      jax.ShapeDtypeStruct((nr, 32), jnp.float32)]

    return _pallas_call(
        body,
        grid=(n // _BT,),
        in_specs=[
            pl.BlockSpec((_BT, dout), lambda i: (i, 0)),
            pl.BlockSpec((2, dout), lambda i: (0, 0)),
            pl.BlockSpec((1, dout), lambda i: (0, 0)),
            pl.BlockSpec((1, dout), lambda i: (0, 0)),
            pl.BlockSpec((_BT, 8), lambda i: (i, 0)),
        ],
        out_specs=out_specs,
        out_shape=out_shape,
    )(p, st, g.reshape(1, -1), bt.reshape(1, -1), dis8)


def kernel(x, pos, normals, edge_index,
           W1, b1, a1, g1, bt1,
           W2, b2, a2, g2, bt2,
           W3, b3, a3, g3, bt3,
           W4, b4, a4, g4, bt4):
    n = x.shape[0]
    e = edge_index.shape[1]
    nr = _round_up(n + 1, _NS * _ZB)
    ep = _round_up(e, _NC * _NS * _RB)
    epr = ep // _RB

    src = edge_index[0].astype(jnp.int32)
    dst = edge_index[1].astype(jnp.int32)
    padk = ep - e
    phantom = jnp.full((padk,), n, jnp.int32)
    srcp = jnp.concatenate([src, phantom]).reshape(epr, _RB)
    dstp = jnp.concatenate([dst, phantom]).reshape(epr, _RB)
    sd = jnp.stack([srcp, dstp], axis=1)

    deg0, deg1 = _sc_degree(nr, epr, srcp)

    h16 = jnp.pad(jnp.concatenate([x, pos, normals], axis=1),
                  ((0, 0), (0, 7)))
    dis8, th0 = _tc_prep(h16, deg0, deg1, n, nr)

    s1a, s1b = _sc_segsum(th0, th0, sd, nr, epr, 16, False)
    th1 = _tc_th1_16(s1a, s1b, dis8, n, nr)
    s2a, s2b = _sc_segsum(th1, th1, sd, nr, epr, 16, False)
    w1p = jnp.pad(W1, ((0, 0), (0, 7), (0, 0)))
    yp = _tc_cheb_a(h16, s1a, s1b, dis8, w1p[0] - w1p[2], w1p[1],
                    b1.reshape(1, -1), n, True)
    p, st = _tc_cheb_b(yp, s2a, s2b, dis8, w1p[2],
                       a1.reshape(1, 1), n, True)
    tx0, tha, thb = _tc_bn(p, st, g1, bt1, dis8, n, nr, False)

    for (W, b, a, g, bt, last) in (
            (W2, b2, a2, g2, bt2, False),
            (W3, b3, a3, g3, bt3, False),
            (W4, b4, a4, g4, bt4, True)):
        s1a, s1b = _sc_segsum(tha, thb, sd, nr, epr, 32, True)
        th1a, th1b = _tc_th1_64(s1a, s1b, dis8, n, nr)
        s2a, s2b = _sc_segsum(th1a, th1b, sd, nr, epr, 32, True)
        yp = _tc_cheb_a(tx0, s1a, s1b, dis8, W[0] - W[2], W[1],
                        b.reshape(1, -1), n, False)
        p, st = _tc_cheb_b(yp, s2a, s2b, dis8, W[2],
                           a.reshape(1, 1), n, False)
        bn = _tc_bn(p, st, g, bt, dis8, n, nr, last)
        if last:
            return bn[0]
        tx0, tha, thb = bn


# speedup vs baseline: 1.2316x; 1.0357x over previous
import functools

import jax
import jax.numpy as jnp
from jax import lax
from jax.experimental import pallas as pl
from jax.experimental.pallas import tpu as pltpu
from jax.experimental.pallas import tpu_sc as plsc

_NC = 2
_NS = 16
_RB = 256
_ZB = 128

_pallas_call = pl.pallas_call
_BT = 5000
_SC_PARAMS = pltpu.CompilerParams(use_tc_tiling_on_sc=False)


def _round_up(v, m):
    return (v + m - 1) // m * m


def _sc_degree(nr, epr, src_rows):
    rt = epr // (_NS * _NC)
    nrt = nr // _NS
    nzc = nrt // _ZB
    mesh = plsc.VectorSubcoreMesh(core_axis_name="c", subcore_axis_name="s",
                                  num_cores=_NC)

    def body(zeros_h, ones_h, src_h, out0, out1, accum, sidx, obuf, zbuf,
             ssem0, ssem1):
        c = lax.axis_index("c")
        s = lax.axis_index("s")
        pltpu.sync_copy(zeros_h, zbuf)

        def zstep(j, carry):
            pltpu.sync_copy(zbuf, accum.at[pl.ds(s * nrt + j * _ZB, _ZB)])
            return carry

        lax.fori_loop(0, nzc, zstep, 0)
        plsc.subcore_barrier()

        pltpu.sync_copy(ones_h, obuf)
        ebase = (c * _NS + s) * rt
        ssems = (ssem0, ssem1)

        def gstart(j, b):
            pltpu.sync_copy(src_h.at[j], sidx.at[b])
            pltpu.async_copy(obuf, accum.at[sidx.at[b]], ssems[b], add=True)

        def swait(b):
            pltpu.make_async_copy(obuf, accum.at[sidx.at[b]],
                                  ssems[b]).wait()

        gstart(ebase, 0)
        gstart(ebase + 1, 1)

        def estep(u, carry):
            i0 = ebase + 2 * u + 2
            swait(0)
            gstart(i0, 0)
            swait(1)
            gstart(i0 + 1, 1)
            return carry

        lax.fori_loop(0, (rt - 2) // 2, estep, 0)
        swait(0)
        swait(1)
        plsc.subcore_barrier()

        @pl.when(c == 0)
        def _():
            pltpu.sync_copy(accum.at[pl.ds(s * nrt, nrt)],
                            out0.at[pl.ds(s * nrt, nrt)])

        @pl.when(c == 1)
        def _():
            pltpu.sync_copy(accum.at[pl.ds(s * nrt, nrt)],
                            out1.at[pl.ds(s * nrt, nrt)])

    fn = pl.kernel(
        body,
        out_type=[jax.ShapeDtypeStruct((nr, 8), jnp.float32),
                  jax.ShapeDtypeStruct((nr, 8), jnp.float32)],
        mesh=mesh,
        scratch_types=[
            pltpu.VMEM_SHARED((nr, 8), jnp.float32),
            pltpu.VMEM((2, _RB), jnp.int32),
            pltpu.VMEM((_RB, 8), jnp.float32),
            pltpu.VMEM((_ZB, 8), jnp.float32),
            pltpu.SemaphoreType.DMA,
            pltpu.SemaphoreType.DMA,
        ],
        compiler_params=_SC_PARAMS,
    )
    zeros_h = jnp.zeros((_ZB, 8), jnp.float32)
    ones_h = jnp.ones((_RB, 8), jnp.float32)
    return fn(zeros_h, ones_h, src_rows)


def _sc_segsum(tab0, tab1, sd_rows, nr, epr, width, feature_split):
    rt = epr // _NS if feature_split else epr // (_NS * _NC)
    nrt = nr // _NS
    nzc = nrt // _ZB
    mesh = plsc.VectorSubcoreMesh(core_axis_name="c", subcore_axis_name="s",
                                  num_cores=_NC)

    def body(zeros_h, tab0_h, tab1_h, sd_h, out0, out1,
             accum, sd, rows, zbuf, gsem0, gsem1, ssem0, ssem1):
        c = lax.axis_index("c")
        s = lax.axis_index("s")
        pltpu.sync_copy(zeros_h, zbuf)

        def zstep(j, carry):
            pltpu.sync_copy(zbuf, accum.at[pl.ds(s * nrt + j * _ZB, _ZB)])
            return carry

        lax.fori_loop(0, nzc, zstep, 0)
        plsc.subcore_barrier()

        ebase = (s * rt) if feature_split else ((c * _NS + s) * rt)
        gsems = (gsem0, gsem1)
        ssems = (ssem0, ssem1)

        def gstart(j, b):
            pltpu.sync_copy(sd_h.at[j], sd.at[b])
            if feature_split:
                @pl.when(c == 0)
                def _():
                    pltpu.async_copy(tab0_h.at[sd.at[b, 0]], rows.at[b],
                                     gsems[b])

                @pl.when(c == 1)
                def _():
                    pltpu.async_copy(tab1_h.at[sd.at[b, 0]], rows.at[b],
                                     gsems[b])
            else:
                pltpu.async_copy(tab0_h.at[sd.at[b, 0]], rows.at[b],
                                 gsems[b])

        def gwait(b):
            pltpu.make_async_copy(tab0_h.at[sd.at[b, 0]], rows.at[b],
                                  gsems[b]).wait()

        def sstart(b):
            pltpu.async_copy(rows.at[b], accum.at[sd.at[b, 1]], ssems[b],
                             add=True)

        def swait(b):
            pltpu.make_async_copy(rows.at[b], accum.at[sd.at[b, 1]],
                                  ssems[b]).wait()

        gstart(ebase, 0)
        gstart(ebase + 1, 1)
        gwait(0)
        sstart(0)

        def mstep(u, carry):
            i0 = ebase + 2 * u + 2
            swait(0)
            gstart(i0, 0)
            gwait(1)
            sstart(1)
            swait(1)
            gstart(i0 + 1, 1)
            gwait(0)
            sstart(0)
            return carry

        lax.fori_loop(0, (rt - 2) // 2, mstep, 0)
        gwait(1)
        sstart(1)
        swait(0)
        swait(1)

        plsc.subcore_barrier()

        @pl.when(c == 0)
        def _():
            pltpu.sync_copy(accum.at[pl.ds(s * nrt, nrt)],
                            out0.at[pl.ds(s * nrt, nrt)])

        @pl.when(c == 1)
        def _():
            pltpu.sync_copy(accum.at[pl.ds(s * nrt, nrt)],
                            out1.at[pl.ds(s * nrt, nrt)])

    fn = pl.kernel(
        body,
        out_type=[jax.ShapeDtypeStruct((nr, width), jnp.float32),
                  jax.ShapeDtypeStruct((nr, width), jnp.float32)],
        mesh=mesh,
        scratch_types=[
            pltpu.VMEM_SHARED((nr, width), jnp.float32),
            pltpu.VMEM((2, 2, _RB), jnp.int32),
            pltpu.VMEM((2, _RB, width), jnp.float32),
            pltpu.VMEM((_ZB, width), jnp.float32),
            pltpu.SemaphoreType.DMA,
            pltpu.SemaphoreType.DMA,
            pltpu.SemaphoreType.DMA,
            pltpu.SemaphoreType.DMA,
        ],
        compiler_params=_SC_PARAMS,
    )
    zeros_h = jnp.zeros((_ZB, width), jnp.float32)
    return fn(zeros_h, tab0, tab1, sd_rows)


def _tc_prep(h16, d0, d1, n, nr):
    def body(h_ref, d0_ref, d1_ref, dis_ref, th_ref):
        deg = d0_ref[:, 0:1] + d1_ref[:, 0:1]
        dis = jnp.where(deg > 0, lax.rsqrt(jnp.maximum(deg, 1e-12)), 0.0)
        dis_ref[...] = jnp.broadcast_to(dis, (_BT, 8))
        th_ref[...] = h_ref[...] * dis

    return _pallas_call(
        body,
        grid=(n // _BT,),
        in_specs=[
            pl.BlockSpec((_BT, 16), lambda i: (i, 0)),
            pl.BlockSpec((_BT, 8), lambda i: (i, 0)),
            pl.BlockSpec((_BT, 8), lambda i: (i, 0)),
        ],
        out_specs=[
            pl.BlockSpec((_BT, 8), lambda i: (i, 0)),
            pl.BlockSpec((_BT, 16), lambda i: (i, 0)),
        ],
        out_shape=[
            jax.ShapeDtypeStruct((n, 8), jnp.float32),
            jax.ShapeDtypeStruct((nr, 16), jnp.float32),
        ],
    )(h16, d0, d1)


def _tc_th1_16(s1a, s1b, dis8, n, nr):
    def body(a_ref, b_ref, dis_ref, th_ref):
        dis = dis_ref[:, 0:1]
        th_ref[...] = (-dis * dis) * (a_ref[...] + b_ref[...])

    return _pallas_call(
        body,
        grid=(n // _BT,),
        in_specs=[
            pl.BlockSpec((_BT, 16), lambda i: (i, 0)),
            pl.BlockSpec((_BT, 16), lambda i: (i, 0)),
            pl.BlockSpec((_BT, 8), lambda i: (i, 0)),
        ],
        out_specs=pl.BlockSpec((_BT, 16), lambda i: (i, 0)),
        out_shape=jax.ShapeDtypeStruct((nr, 16), jnp.float32),
    )(s1a, s1b, dis8)


def _tc_th1_64(s1a, s1b, dis8, n, nr):
    def body(a_ref, b_ref, dis_ref, ta_ref, tb_ref):
        dis = dis_ref[:, 0:1]
        nd2 = -dis * dis
        ta_ref[...] = nd2 * a_ref[...]
        tb_ref[...] = nd2 * b_ref[...]

    return _pallas_call(
        body,
        grid=(n // _BT,),
        in_specs=[
            pl.BlockSpec((_BT, 32), lambda i: (i, 0)),
            pl.BlockSpec((_BT, 32), lambda i: (i, 0)),
            pl.BlockSpec((_BT, 8), lambda i: (i, 0)),
        ],
        out_specs=[
            pl.BlockSpec((_BT, 32), lambda i: (i, 0)),
            pl.BlockSpec((_BT, 32), lambda i: (i, 0)),
        ],
        out_shape=[
            jax.ShapeDtypeStruct((nr, 32), jnp.float32),
            jax.ShapeDtypeStruct((nr, 32), jnp.float32),
        ],
    )(s1a, s1b, dis8)


def _tc_cheb(tx0, s1a, s1b, s2a, s2b, dis8, w0, w1, w2, b, a, n,
             partial_mode):
    din = w0.shape[0]
    dout = w0.shape[1]
    hw = s1a.shape[1]

    def body(tx0_ref, s1a_ref, s1b_ref, s2a_ref, s2b_ref, dis_ref,
             w0_ref, w1_ref, w2_ref, b_ref, a_ref, p_ref, st_ref):
        i = pl.program_id(0)
        dis = dis_ref[:, 0:1]
        if partial_mode:
            s1 = s1a_ref[...] + s1b_ref[...]
            s2 = s2a_ref[...] + s2b_ref[...]
        else:
            s1 = jnp.concatenate([s1a_ref[...], s1b_ref[...]], axis=1)
            s2 = jnp.concatenate([s2a_ref[...], s2b_ref[...]], axis=1)
        tx0v = tx0_ref[...]
        tx1 = (-dis) * s1
        tx2 = (-2.0 * dis) * s2 - tx0v
        y = (jnp.dot(tx0v, w0_ref[...], preferred_element_type=jnp.float32)
             + jnp.dot(tx1, w1_ref[...], preferred_element_type=jnp.float32)
             + jnp.dot(tx2, w2_ref[...], preferred_element_type=jnp.float32)
             + b_ref[...])
        aa = a_ref[0, 0]
        p = jnp.where(y > 0, y, aa * y)
        p_ref[...] = p

        @pl.when(i == 0)
        def _():
            st_ref[...] = jnp.zeros_like(st_ref)

        st_ref[0:1, :] += jnp.sum(p, axis=0, keepdims=True)
        st_ref[1:2, :] += jnp.sum(p * p, axis=0, keepdims=True)

    return _pallas_call(
        body,
        grid=(n // _BT,),
        in_specs=[
            pl.BlockSpec((_BT, din), lambda i: (i, 0)),
            pl.BlockSpec((_BT, hw), lambda i: (i, 0)),
            pl.BlockSpec((_BT, hw), lambda i: (i, 0)),
            pl.BlockSpec((_BT, hw), lambda i: (i, 0)),
            pl.BlockSpec((_BT, hw), lambda i: (i, 0)),
            pl.BlockSpec((_BT, 8), lambda i: (i, 0)),
            pl.BlockSpec((din, dout), lambda i: (0, 0)),
            pl.BlockSpec((din, dout), lambda i: (0, 0)),
            pl.BlockSpec((din, dout), lambda i: (0, 0)),
            pl.BlockSpec((1, dout), lambda i: (0, 0)),
            pl.BlockSpec((1, 1), lambda i: (0, 0)),
        ],
        out_specs=[
            pl.BlockSpec((_BT, dout), lambda i: (i, 0)),
            pl.BlockSpec((2, dout), lambda i: (0, 0)),
        ],
        out_shape=[
            jax.ShapeDtypeStruct((n, dout), jnp.float32),
            jax.ShapeDtypeStruct((2, dout), jnp.float32),
        ],
    )(tx0, s1a, s1b, s2a, s2b, dis8, w0, w1, w2, b, a)


def _tc_bn(p, st, g, bt, dis8, n, nr, last):
    dout = p.shape[1]
    inv_n = 1.0 / n

    def body(p_ref, st_ref, g_ref, bt_ref, dis_ref, *outs):
        mean = st_ref[0:1, :] * inv_n
        var = st_ref[1:2, :] * inv_n - mean * mean
        scale = lax.rsqrt(var + 1e-5) * g_ref[...]
        y = (p_ref[...] - mean) * scale + bt_ref[...]
        outs[0][...] = y
        if not last:
            dis = dis_ref[:, 0:1]
            th = dis * y
            outs[1][...] = th[:, :32]
            outs[2][...] = th[:, 32:]

    out_specs = [pl.BlockSpec((_BT, dout), lambda i: (i, 0))]
    out_shape = [jax.ShapeDtypeStruct((n, dout), jnp.float32)]
    if not last:
        out_specs += [pl.BlockSpec((_BT, 32), lambda i: (i, 0)),
                      pl.BlockSpec((_BT, 32), lambda i: (i, 0))]
        out_shape += [jax.ShapeDtypeStruct((nr, 32), jnp.float32),
                      jax.ShapeDtypeStruct((nr, 32), jnp.float32)]

    return _pallas_call(
        body,
        grid=(n // _BT,),
        in_specs=[
            pl.BlockSpec((_BT, dout), lambda i: (i, 0)),
            pl.BlockSpec((2, dout), lambda i: (0, 0)),
            pl.BlockSpec((1, dout), lambda i: (0, 0)),
            pl.BlockSpec((1, dout), lambda i: (0, 0)),
            pl.BlockSpec((_BT, 8), lambda i: (i, 0)),
        ],
        out_specs=out_specs,
        out_shape=out_shape,
    )(p, st, g.reshape(1, -1), bt.reshape(1, -1), dis8)


def kernel(x, pos, normals, edge_index,
           W1, b1, a1, g1, bt1,
           W2, b2, a2, g2, bt2,
           W3, b3, a3, g3, bt3,
           W4, b4, a4, g4, bt4):
    n = x.shape[0]
    e = edge_index.shape[1]
    nr = _round_up(n + 1, _NS * _ZB)
    ep = _round_up(e, _NC * _NS * _RB)
    epr = ep // _RB

    src = edge_index[0].astype(jnp.int32)
    dst = edge_index[1].astype(jnp.int32)
    padk = ep - e
    phantom = jnp.full((padk,), n, jnp.int32)
    srcp = jnp.concatenate([src, phantom]).reshape(epr, _RB)
    dstp = jnp.concatenate([dst, phantom]).reshape(epr, _RB)
    sd = jnp.stack([srcp, dstp], axis=1)

    deg0, deg1 = _sc_degree(nr, epr, srcp)

    h16 = jnp.pad(jnp.concatenate([x, pos, normals], axis=1),
                  ((0, 0), (0, 7)))
    dis8, th0 = _tc_prep(h16, deg0, deg1, n, nr)

    s1a, s1b = _sc_segsum(th0, th0, sd, nr, epr, 16, False)
    th1 = _tc_th1_16(s1a, s1b, dis8, n, nr)
    s2a, s2b = _sc_segsum(th1, th1, sd, nr, epr, 16, False)
    w1p = jnp.pad(W1, ((0, 0), (0, 7), (0, 0)))
    p, st = _tc_cheb(h16, s1a, s1b, s2a, s2b, dis8,
                     w1p[0], w1p[1], w1p[2],
                     b1.reshape(1, -1), a1.reshape(1, 1), n, True)
    tx0, tha, thb = _tc_bn(p, st, g1, bt1, dis8, n, nr, False)

    for (W, b, a, g, bt, last) in (
            (W2, b2, a2, g2, bt2, False),
            (W3, b3, a3, g3, bt3, False),
            (W4, b4, a4, g4, bt4, True)):
        s1a, s1b = _sc_segsum(tha, thb, sd, nr, epr, 32, True)
        th1a, th1b = _tc_th1_64(s1a, s1b, dis8, n, nr)
        s2a, s2b = _sc_segsum(th1a, th1b, sd, nr, epr, 32, True)
        p, st = _tc_cheb(tx0, s1a, s1b, s2a, s2b, dis8,
                         W[0], W[1], W[2],
                         b.reshape(1, -1), a.reshape(1, 1), n, False)
        bn = _tc_bn(p, st, g, bt, dis8, n, nr, last)
        if last:
            return bn[0]
        tx0, tha, thb = bn
